# Initial kernel scaffold; baseline (speedup 1.0000x reference)
import jax
import jax.numpy as jnp
from jax.experimental import pallas as pl


def kernel(observation, belief_counts):
    raise NotImplementedError("write your pallas kernel here")


# speedup vs baseline: 8.8050x; 8.8050x over previous
import functools

import jax
import jax.numpy as jnp
from jax.experimental import pallas as pl

_DIM = 1048576
_NUM_BINS = 16
_BLOCK = 8192


def _body(obs_ref, bel_ref, ig_ref, hp_ref, hq_ref, mig_ref, ev_ref):
    i = pl.program_id(0)
    n = pl.num_programs(0)

    c = bel_ref[...]
    t = c.T
    total = jnp.sum(t, axis=0, keepdims=True)
    tsafe = jnp.maximum(total, 1e-8)
    lc = jnp.log2(t + 1e-10)
    s = jnp.sum(t * lc, axis=0, keepdims=True)
    log_t = jnp.log2(tsafe)
    hp = (total * log_t - s) / tsafe

    m = jnp.mean(obs_ref[...], axis=0, keepdims=True)
    sig = jax.nn.sigmoid(m)
    bin_i = jnp.clip((sig * (_NUM_BINS - 1)).astype(jnp.int32), 0, _NUM_BINS - 1)

    rows = jax.lax.broadcasted_iota(jnp.int32, t.shape, 0)
    onehot = (rows == bin_i).astype(jnp.float32)
    cb = jnp.sum(t * onehot, axis=0, keepdims=True)

    lb = jnp.log2(cb + 1e-10)
    lb1 = jnp.log2(cb + 1.0 + 1e-10)
    t2 = total + 1.0
    t2safe = jnp.maximum(t2, 1e-8)
    log_t2 = jnp.log2(t2safe)
    s2 = s - cb * lb + (cb + 1.0) * lb1
    hq = (t2 * log_t2 - s2) / t2safe

    kl = (cb + 1.0) * (lb1 - lb) / t2safe + log_t - log_t2
    ig = jnp.maximum(kl, 0.0)

    ig_ref[...] = ig
    hp_ref[...] = hp
    hq_ref[...] = hq

    @pl.when(i == 0)
    def _init():
        mig_ref[...] = jnp.zeros((1, 1), jnp.float32)

    mig_ref[...] += jnp.sum(ig).reshape(1, 1)

    @pl.when(i == n - 1)
    def _fin():
        mig = mig_ref[...] / _DIM
        mig_ref[...] = mig
        ev_ref[...] = jax.nn.sigmoid(mig * 50.0 - 1.0)


@functools.partial(jax.jit, static_argnames=())
def kernel(observation, belief_counts):
    grid = (_DIM // _BLOCK,)
    out = pl.pallas_call(
        _body,
        grid=grid,
        in_specs=[
            pl.BlockSpec((observation.shape[0], _BLOCK), lambda i: (0, i)),
            pl.BlockSpec((_BLOCK, _NUM_BINS), lambda i: (i, 0)),
        ],
        out_specs=[
            pl.BlockSpec((1, _BLOCK), lambda i: (0, i)),
            pl.BlockSpec((1, _BLOCK), lambda i: (0, i)),
            pl.BlockSpec((1, _BLOCK), lambda i: (0, i)),
            pl.BlockSpec((1, 1), lambda i: (0, 0)),
            pl.BlockSpec((1, 1), lambda i: (0, 0)),
        ],
        out_shape=[
            jax.ShapeDtypeStruct((1, _DIM), jnp.float32),
            jax.ShapeDtypeStruct((1, _DIM), jnp.float32),
            jax.ShapeDtypeStruct((1, _DIM), jnp.float32),
            jax.ShapeDtypeStruct((1, 1), jnp.float32),
            jax.ShapeDtypeStruct((1, 1), jnp.float32),
        ],
    )(observation, belief_counts)
    ig, hp, hq, mig, ev = out
    return (
        ig.reshape(_DIM),
        mig.reshape(()),
        hp.reshape(_DIM),
        hq.reshape(_DIM),
        ev.reshape(()),
    )


# speedup vs baseline: 9.7465x; 1.1069x over previous
import functools

import jax
import jax.numpy as jnp
from jax.experimental import pallas as pl

_DIM = 1048576
_NUM_BINS = 16
_BLOCK = 16384


def _body(obs_ref, bel_ref, ig_ref, hp_ref, hq_ref, mig_ref, ev_ref):
    i = pl.program_id(0)
    n = pl.num_programs(0)

    c = bel_ref[...]
    t = c.T
    total = jnp.sum(t, axis=0, keepdims=True)
    tsafe = jnp.maximum(total, 1e-8)
    lc = jnp.log2(t + 1e-10)
    s = jnp.sum(t * lc, axis=0, keepdims=True)
    log_t = jnp.log2(tsafe)
    hp = (total * log_t - s) / tsafe

    m = jnp.mean(obs_ref[...], axis=0, keepdims=True)
    sig = jax.nn.sigmoid(m)
    bin_i = jnp.clip((sig * (_NUM_BINS - 1)).astype(jnp.int32), 0, _NUM_BINS - 1)

    rows = jax.lax.broadcasted_iota(jnp.int32, t.shape, 0)
    onehot = (rows == bin_i).astype(jnp.float32)
    cb = jnp.sum(t * onehot, axis=0, keepdims=True)

    lb = jnp.log2(cb + 1e-10)
    lb1 = jnp.log2(cb + 1.0 + 1e-10)
    t2 = total + 1.0
    t2safe = jnp.maximum(t2, 1e-8)
    log_t2 = jnp.log2(t2safe)
    s2 = s - cb * lb + (cb + 1.0) * lb1
    hq = (t2 * log_t2 - s2) / t2safe

    kl = (cb + 1.0) * (lb1 - lb) / t2safe + log_t - log_t2
    ig = jnp.maximum(kl, 0.0)

    ig_ref[...] = ig
    hp_ref[...] = hp
    hq_ref[...] = hq

    @pl.when(i == 0)
    def _init():
        mig_ref[...] = jnp.zeros((1, 1), jnp.float32)

    mig_ref[...] += jnp.sum(ig).reshape(1, 1)

    @pl.when(i == n - 1)
    def _fin():
        mig = mig_ref[...] / _DIM
        mig_ref[...] = mig
        ev_ref[...] = jax.nn.sigmoid(mig * 50.0 - 1.0)


@functools.partial(jax.jit, static_argnames=())
def kernel(observation, belief_counts):
    grid = (_DIM // _BLOCK,)
    out = pl.pallas_call(
        _body,
        grid=grid,
        in_specs=[
            pl.BlockSpec((observation.shape[0], _BLOCK), lambda i: (0, i)),
            pl.BlockSpec((_BLOCK, _NUM_BINS), lambda i: (i, 0)),
        ],
        out_specs=[
            pl.BlockSpec((1, _BLOCK), lambda i: (0, i)),
            pl.BlockSpec((1, _BLOCK), lambda i: (0, i)),
            pl.BlockSpec((1, _BLOCK), lambda i: (0, i)),
            pl.BlockSpec((1, 1), lambda i: (0, 0)),
            pl.BlockSpec((1, 1), lambda i: (0, 0)),
        ],
        out_shape=[
            jax.ShapeDtypeStruct((1, _DIM), jnp.float32),
            jax.ShapeDtypeStruct((1, _DIM), jnp.float32),
            jax.ShapeDtypeStruct((1, _DIM), jnp.float32),
            jax.ShapeDtypeStruct((1, 1), jnp.float32),
            jax.ShapeDtypeStruct((1, 1), jnp.float32),
        ],
    )(observation, belief_counts)
    ig, hp, hq, mig, ev = out
    return (
        ig.reshape(_DIM),
        mig.reshape(()),
        hp.reshape(_DIM),
        hq.reshape(_DIM),
        ev.reshape(()),
    )


# speedup vs baseline: 10.2010x; 1.0466x over previous
import functools

import jax
import jax.numpy as jnp
from jax.experimental import pallas as pl

_DIM = 1048576
_NUM_BINS = 16
_BLOCK = 32768


def _body(obs_ref, bel_ref, ig_ref, hp_ref, hq_ref, mig_ref, ev_ref):
    i = pl.program_id(0)
    n = pl.num_programs(0)

    c = bel_ref[...]
    t = c.T
    total = jnp.sum(t, axis=0, keepdims=True)
    tsafe = jnp.maximum(total, 1e-8)
    lc = jnp.log2(t + 1e-10)
    s = jnp.sum(t * lc, axis=0, keepdims=True)
    log_t = jnp.log2(tsafe)
    hp = (total * log_t - s) / tsafe

    m = jnp.mean(obs_ref[...], axis=0, keepdims=True)
    sig = jax.nn.sigmoid(m)
    bin_i = jnp.clip((sig * (_NUM_BINS - 1)).astype(jnp.int32), 0, _NUM_BINS - 1)

    rows = jax.lax.broadcasted_iota(jnp.int32, t.shape, 0)
    onehot = (rows == bin_i).astype(jnp.float32)
    cb = jnp.sum(t * onehot, axis=0, keepdims=True)

    lb = jnp.log2(cb + 1e-10)
    lb1 = jnp.log2(cb + 1.0 + 1e-10)
    t2 = total + 1.0
    t2safe = jnp.maximum(t2, 1e-8)
    log_t2 = jnp.log2(t2safe)
    s2 = s - cb * lb + (cb + 1.0) * lb1
    hq = (t2 * log_t2 - s2) / t2safe

    kl = (cb + 1.0) * (lb1 - lb) / t2safe + log_t - log_t2
    ig = jnp.maximum(kl, 0.0)

    ig_ref[...] = ig
    hp_ref[...] = hp
    hq_ref[...] = hq

    @pl.when(i == 0)
    def _init():
        mig_ref[...] = jnp.zeros((1, 1), jnp.float32)

    mig_ref[...] += jnp.sum(ig).reshape(1, 1)

    @pl.when(i == n - 1)
    def _fin():
        mig = mig_ref[...] / _DIM
        mig_ref[...] = mig
        ev_ref[...] = jax.nn.sigmoid(mig * 50.0 - 1.0)


@functools.partial(jax.jit, static_argnames=())
def kernel(observation, belief_counts):
    grid = (_DIM // _BLOCK,)
    out = pl.pallas_call(
        _body,
        grid=grid,
        in_specs=[
            pl.BlockSpec((observation.shape[0], _BLOCK), lambda i: (0, i)),
            pl.BlockSpec((_BLOCK, _NUM_BINS), lambda i: (i, 0)),
        ],
        out_specs=[
            pl.BlockSpec((1, _BLOCK), lambda i: (0, i)),
            pl.BlockSpec((1, _BLOCK), lambda i: (0, i)),
            pl.BlockSpec((1, _BLOCK), lambda i: (0, i)),
            pl.BlockSpec((1, 1), lambda i: (0, 0)),
            pl.BlockSpec((1, 1), lambda i: (0, 0)),
        ],
        out_shape=[
            jax.ShapeDtypeStruct((1, _DIM), jnp.float32),
            jax.ShapeDtypeStruct((1, _DIM), jnp.float32),
            jax.ShapeDtypeStruct((1, _DIM), jnp.float32),
            jax.ShapeDtypeStruct((1, 1), jnp.float32),
            jax.ShapeDtypeStruct((1, 1), jnp.float32),
        ],
    )(observation, belief_counts)
    ig, hp, hq, mig, ev = out
    return (
        ig.reshape(_DIM),
        mig.reshape(()),
        hp.reshape(_DIM),
        hq.reshape(_DIM),
        ev.reshape(()),
    )


# speedup vs baseline: 36.4053x; 3.5688x over previous
import functools

import jax
import jax.numpy as jnp
from jax.experimental import pallas as pl

_DIM = 1048576
_NUM_BINS = 16
_BLOCK = 65536


def _body(obs_ref, bel_ref, ig_ref, hp_ref, hq_ref, mig_ref, ev_ref):
    i = pl.program_id(0)
    n = pl.num_programs(0)

    t = bel_ref[...]
    total = jnp.sum(t, axis=0, keepdims=True)
    tsafe = jnp.maximum(total, 1e-8)
    lc = jnp.log2(t + 1e-10)
    s = jnp.sum(t * lc, axis=0, keepdims=True)
    log_t = jnp.log2(tsafe)
    hp = (total * log_t - s) / tsafe

    m = jnp.mean(obs_ref[...], axis=0, keepdims=True)
    sig = jax.nn.sigmoid(m)
    bin_i = jnp.clip((sig * (_NUM_BINS - 1)).astype(jnp.int32), 0, _NUM_BINS - 1)

    rows = jax.lax.broadcasted_iota(jnp.int32, t.shape, 0)
    onehot = (rows == bin_i).astype(jnp.float32)
    cb = jnp.sum(t * onehot, axis=0, keepdims=True)

    lb = jnp.log2(cb + 1e-10)
    lb1 = jnp.log2(cb + 1.0 + 1e-10)
    t2 = total + 1.0
    t2safe = jnp.maximum(t2, 1e-8)
    log_t2 = jnp.log2(t2safe)
    s2 = s - cb * lb + (cb + 1.0) * lb1
    hq = (t2 * log_t2 - s2) / t2safe

    kl = (cb + 1.0) * (lb1 - lb) / t2safe + log_t - log_t2
    ig = jnp.maximum(kl, 0.0)

    ig_ref[...] = ig
    hp_ref[...] = hp
    hq_ref[...] = hq

    @pl.when(i == 0)
    def _init():
        mig_ref[...] = jnp.zeros((1, 1), jnp.float32)

    mig_ref[...] += jnp.sum(ig).reshape(1, 1)

    @pl.when(i == n - 1)
    def _fin():
        mig = mig_ref[...] / _DIM
        mig_ref[...] = mig
        ev_ref[...] = jax.nn.sigmoid(mig * 50.0 - 1.0)


@functools.partial(jax.jit, static_argnames=())
def kernel(observation, belief_counts):
    grid = (_DIM // _BLOCK,)
    out = pl.pallas_call(
        _body,
        grid=grid,
        in_specs=[
            pl.BlockSpec((observation.shape[0], _BLOCK), lambda i: (0, i)),
            pl.BlockSpec((_NUM_BINS, _BLOCK), lambda i: (0, i)),
        ],
        out_specs=[
            pl.BlockSpec((1, _BLOCK), lambda i: (0, i)),
            pl.BlockSpec((1, _BLOCK), lambda i: (0, i)),
            pl.BlockSpec((1, _BLOCK), lambda i: (0, i)),
            pl.BlockSpec((1, 1), lambda i: (0, 0)),
            pl.BlockSpec((1, 1), lambda i: (0, 0)),
        ],
        out_shape=[
            jax.ShapeDtypeStruct((1, _DIM), jnp.float32),
            jax.ShapeDtypeStruct((1, _DIM), jnp.float32),
            jax.ShapeDtypeStruct((1, _DIM), jnp.float32),
            jax.ShapeDtypeStruct((1, 1), jnp.float32),
            jax.ShapeDtypeStruct((1, 1), jnp.float32),
        ],
    )(observation, belief_counts.T)
    ig, hp, hq, mig, ev = out
    return (
        ig.reshape(_DIM),
        mig.reshape(()),
        hp.reshape(_DIM),
        hq.reshape(_DIM),
        ev.reshape(()),
    )
